# baseline (speedup 1.0000x reference)
import functools

import jax
import jax.numpy as jnp
from jax import lax
from jax.experimental import pallas as pl
from jax.experimental.pallas import tpu as pltpu
from jax.experimental.pallas import tpu_sc as plsc

V = 100000
E = 128
H = 512
B = 8
S = 64
T = B * S
G = 4 * H


def _sc_gather(table, idx_flat):
    info = plsc.get_sparse_core_info()
    nc, ns = info.num_cores, info.num_subcores
    nw = nc * ns
    bpw = T // nw
    mesh = plsc.VectorSubcoreMesh(core_axis_name="c", subcore_axis_name="s")

    @functools.partial(
        pl.kernel,
        mesh=mesh,
        out_type=jax.ShapeDtypeStruct((T, E), jnp.float32),
        scratch_types=[
            pltpu.VMEM((bpw,), jnp.int32),
            pltpu.VMEM((bpw, E), jnp.float32),
            pltpu.SemaphoreType.DMA,
        ],
    )
    def k(table_hbm, idx_hbm, out_hbm, idx_v, rows_v, sem):
        wid = lax.axis_index("s") * nc + lax.axis_index("c")
        base = wid * bpw
        pltpu.sync_copy(idx_hbm.at[pl.ds(base, bpw)], idx_v)
        pltpu.async_copy(table_hbm.at[idx_v], rows_v, sem).wait()
        pltpu.sync_copy(rows_v, out_hbm.at[pl.ds(base, bpw)])

    return k(table, idx_flat)


def _lstm_body(x_ref, wih0t, b0, whh0t, whh1t_cat, b1, wp1t, bp1,
               wp2t, bp2, out_ref, pre0_ref, hs_ref):
    pre0_ref[...] = b0[...] + jnp.dot(
        x_ref[...].astype(jnp.bfloat16), wih0t[...],
        preferred_element_type=jnp.float32)

    def step(k, carry):
        h0, c0, h1, c1 = carry
        h0b = h0.astype(jnp.bfloat16)
        a1 = jnp.concatenate([h0b, h1.astype(jnp.bfloat16)], axis=1)
        row = jnp.minimum(k, S - 1) * B
        g0 = pre0_ref[pl.ds(row, B), :] + jnp.dot(
            h0b, whh0t[...], preferred_element_type=jnp.float32)
        g1 = b1[...] + jnp.dot(
            a1, whh1t_cat[...], preferred_element_type=jnp.float32)
        i0 = jax.nn.sigmoid(g0[:, 0:H])
        f0 = jax.nn.sigmoid(g0[:, H:2 * H])
        t0 = jnp.tanh(g0[:, 2 * H:3 * H])
        o0 = jax.nn.sigmoid(g0[:, 3 * H:4 * H])
        i1 = jax.nn.sigmoid(g1[:, 0:H])
        f1 = jax.nn.sigmoid(g1[:, H:2 * H])
        t1 = jnp.tanh(g1[:, 2 * H:3 * H])
        o1 = jax.nn.sigmoid(g1[:, 3 * H:4 * H])
        c0 = f0 * c0 + i0 * t0
        h0 = o0 * jnp.tanh(c0)
        c1n = f1 * c1 + i1 * t1
        h1n = o1 * jnp.tanh(c1n)
        first = k == 0
        z = jnp.zeros((B, H), jnp.float32)
        c1 = jnp.where(first, z, c1n)
        h1 = jnp.where(first, z, h1n)
        hs_ref[:, jnp.maximum(k - 1, 0), :] = h1
        return h0, c0, h1, c1

    z = jnp.zeros((B, H), jnp.float32)
    lax.fori_loop(0, S + 1, step, (z, z, z, z), unroll=5)

    p1 = jnp.tanh(
        jnp.dot(hs_ref[...].reshape(T, H).astype(jnp.bfloat16), wp1t[...],
                preferred_element_type=jnp.float32)
        + bp1[...])
    out_ref[...] = (
        jnp.dot(p1.astype(jnp.bfloat16), wp2t[...],
                preferred_element_type=jnp.float32) + bp2[...])


def _lstm_proj(x_tm, wih0t, b0, whh0t, whh1t_cat, b1, wp1t, bp1, wp2t,
               bp2, interpret=False):
    return pl.pallas_call(
        _lstm_body,
        out_shape=jax.ShapeDtypeStruct((T, E), jnp.float32),
        scratch_shapes=[
            pltpu.VMEM((T, G), jnp.float32),
            pltpu.VMEM((B, S, H), jnp.float32),
        ],
        interpret=interpret,
    )(x_tm, wih0t, b0, whh0t, whh1t_cat, b1, wp1t, bp1, wp2t, bp2)


_TV = 8192


def _logits_body(x_ref, emb_ref, gb_ref, out_ref):
    out_ref[...] = lax.dot_general(
        x_ref[...], emb_ref[...].astype(jnp.bfloat16),
        (((1,), (1,)), ((), ())),
        preferred_element_type=jnp.float32,
    ) + gb_ref[...]


def _logits(x_bm, emb_table, gen_b2d, interpret=False):
    nv = pl.cdiv(V, _TV)
    return pl.pallas_call(
        _logits_body,
        grid=(nv,),
        in_specs=[
            pl.BlockSpec((T, E), lambda i: (0, 0)),
            pl.BlockSpec((_TV, E), lambda i: (i, 0)),
            pl.BlockSpec((1, _TV), lambda i: (0, i)),
        ],
        out_specs=pl.BlockSpec((T, _TV), lambda i: (0, i)),
        out_shape=jax.ShapeDtypeStruct((T, V), jnp.float32),
        interpret=interpret,
    )(x_bm, emb_table, gen_b2d)


def kernel(sentence, emb_table, W_ih0, W_hh0, b_ih0, b_hh0, W_ih1, W_hh1,
           b_ih1, b_hh1, W_p1, b_p1, W_p2, b_p2, gen_b):
    idx_tm = jnp.transpose(sentence).reshape(T).astype(jnp.int32)
    x_tm = _sc_gather(emb_table, idx_tm)

    whh1t_cat = jnp.concatenate(
        [W_ih1.T, W_hh1.T], axis=0).astype(jnp.bfloat16)

    out_bm = _lstm_proj(
        x_tm,
        W_ih0.T.astype(jnp.bfloat16), (b_ih0 + b_hh0).reshape(1, G),
        W_hh0.T.astype(jnp.bfloat16), whh1t_cat,
        (b_ih1 + b_hh1).reshape(1, G),
        W_p1.T.astype(jnp.bfloat16), b_p1.reshape(1, H),
        W_p2.T.astype(jnp.bfloat16), b_p2.reshape(1, E),
    )

    logits = _logits(out_bm.astype(jnp.bfloat16), emb_table,
                     gen_b.reshape(1, V))
    return logits.reshape(B, S, V)


# speedup vs baseline: 1.0562x; 1.0562x over previous
import functools

import jax
import jax.numpy as jnp
from jax import lax
from jax.experimental import pallas as pl
from jax.experimental.pallas import tpu as pltpu
from jax.experimental.pallas import tpu_sc as plsc

V = 100000
E = 128
H = 512
B = 8
S = 64
T = B * S
G = 4 * H

TV = 4096
NV = (V + TV - 1) // TV
LAST = V - (NV - 1) * TV
NBUF = 12


def _sc_gather(table, idx_flat):
    info = plsc.get_sparse_core_info()
    nc, ns = info.num_cores, info.num_subcores
    nw = nc * ns
    bpw = T // nw
    mesh = plsc.VectorSubcoreMesh(core_axis_name="c", subcore_axis_name="s")

    @functools.partial(
        pl.kernel,
        mesh=mesh,
        out_type=jax.ShapeDtypeStruct((T, E), jnp.float32),
        scratch_types=[
            pltpu.VMEM((bpw,), jnp.int32),
            pltpu.VMEM((bpw, E), jnp.float32),
            pltpu.SemaphoreType.DMA,
        ],
    )
    def k(table_hbm, idx_hbm, out_hbm, idx_v, rows_v, sem):
        wid = lax.axis_index("s") * nc + lax.axis_index("c")
        base = wid * bpw
        pltpu.sync_copy(idx_hbm.at[pl.ds(base, bpw)], idx_v)
        pltpu.async_copy(table_hbm.at[idx_v], rows_v, sem).wait()
        pltpu.sync_copy(rows_v, out_hbm.at[pl.ds(base, bpw)])

    return k(table, idx_flat)


def _run_lstm(x_ref, wih0t, b0, whh0t, whh1t_cat, b1, wp1t, bp1, wp2t,
              bp2, pre0_ref, hs_ref, outs_ref):
    pre0_ref[...] = b0[...] + jnp.dot(
        x_ref[...].astype(jnp.bfloat16), wih0t[...],
        preferred_element_type=jnp.float32)

    def step(k, carry):
        h0, c0, h1, c1 = carry
        h0b = h0.astype(jnp.bfloat16)
        a1 = jnp.concatenate([h0b, h1.astype(jnp.bfloat16)], axis=1)
        row = jnp.minimum(k, S - 1) * B
        g0 = pre0_ref[pl.ds(row, B), :] + jnp.dot(
            h0b, whh0t[...], preferred_element_type=jnp.float32)
        g1 = b1[...] + jnp.dot(
            a1, whh1t_cat[...], preferred_element_type=jnp.float32)
        i0 = jax.nn.sigmoid(g0[:, 0:H])
        f0 = jax.nn.sigmoid(g0[:, H:2 * H])
        t0 = jnp.tanh(g0[:, 2 * H:3 * H])
        o0 = jax.nn.sigmoid(g0[:, 3 * H:4 * H])
        i1 = jax.nn.sigmoid(g1[:, 0:H])
        f1 = jax.nn.sigmoid(g1[:, H:2 * H])
        t1 = jnp.tanh(g1[:, 2 * H:3 * H])
        o1 = jax.nn.sigmoid(g1[:, 3 * H:4 * H])
        c0 = f0 * c0 + i0 * t0
        h0 = o0 * jnp.tanh(c0)
        c1n = f1 * c1 + i1 * t1
        h1n = o1 * jnp.tanh(c1n)
        first = k == 0
        z = jnp.zeros((B, H), jnp.float32)
        c1 = jnp.where(first, z, c1n)
        h1 = jnp.where(first, z, h1n)
        hs_ref[:, jnp.maximum(k - 1, 0), :] = h1
        return h0, c0, h1, c1

    z = jnp.zeros((B, H), jnp.float32)
    lax.fori_loop(0, S + 1, step, (z, z, z, z), unroll=5)

    p1 = jnp.tanh(
        jnp.dot(hs_ref[...].reshape(T, H).astype(jnp.bfloat16), wp1t[...],
                preferred_element_type=jnp.float32)
        + bp1[...])
    outs_ref[...] = (jnp.dot(
        p1.astype(jnp.bfloat16), wp2t[...],
        preferred_element_type=jnp.float32) + bp2[...]).astype(jnp.bfloat16)


def _fused_body(x_ref, wih0t, b0, whh0t, whh1t_cat, b1, wp1t, bp1, wp2t,
                bp2, emb_hbm, gb_ref, out_ref,
                pre0_ref, hs_ref, outs_ref, ring_ref, sems):
    i = pl.program_id(0)

    @pl.when(i == 0)
    def _prologue():
        for s in range(NBUF):
            pltpu.make_async_copy(
                emb_hbm.at[pl.ds(s * TV, TV), :], ring_ref.at[s],
                sems.at[s]).start()
        _run_lstm(x_ref, wih0t, b0, whh0t, whh1t_cat, b1, wp1t, bp1,
                  wp2t, bp2, pre0_ref, hs_ref, outs_ref)

    @pl.when(jnp.logical_and(i >= 1, i <= NV - NBUF - 1))
    def _refill():
        j = i - 1 + NBUF
        slot = lax.rem(i - 1, NBUF)
        pltpu.make_async_copy(
            emb_hbm.at[pl.ds(j * TV, TV), :], ring_ref.at[slot],
            sems.at[slot]).start()

    @pl.when(i == NV - NBUF)
    def _refill_last():
        slot = lax.rem(i - 1, NBUF)
        pltpu.make_async_copy(
            emb_hbm.at[pl.ds((NV - 1) * TV, LAST), :],
            ring_ref.at[slot, pl.ds(0, LAST), :], sems.at[slot]).start()

    slot = lax.rem(i, NBUF)

    @pl.when(i < NV - 1)
    def _wait_full():
        pltpu.make_async_copy(
            emb_hbm.at[pl.ds(i * TV, TV), :], ring_ref.at[slot],
            sems.at[slot]).wait()

    @pl.when(i == NV - 1)
    def _wait_last():
        pltpu.make_async_copy(
            emb_hbm.at[pl.ds((NV - 1) * TV, LAST), :],
            ring_ref.at[slot, pl.ds(0, LAST), :], sems.at[slot]).wait()

    out_ref[...] = lax.dot_general(
        outs_ref[...], ring_ref[slot].astype(jnp.bfloat16),
        (((1,), (1,)), ((), ())),
        preferred_element_type=jnp.float32,
    ) + gb_ref[...]


def _fused(x_tm, wih0t, b0, whh0t, whh1t_cat, b1, wp1t, bp1, wp2t, bp2,
           emb_table, gen_b2d):
    full = lambda shape: pl.BlockSpec(shape, lambda i: tuple(
        0 for _ in shape))
    return pl.pallas_call(
        _fused_body,
        grid=(NV,),
        in_specs=[
            full((T, E)),
            full((E, G)), full((1, G)),
            full((H, G)), full((2 * H, G)), full((1, G)),
            full((H, H)), full((1, H)),
            full((H, E)), full((1, E)),
            pl.BlockSpec(memory_space=pl.ANY),
            pl.BlockSpec((1, TV), lambda i: (0, i)),
        ],
        out_specs=pl.BlockSpec((T, TV), lambda i: (0, i)),
        out_shape=jax.ShapeDtypeStruct((T, V), jnp.float32),
        scratch_shapes=[
            pltpu.VMEM((T, G), jnp.float32),
            pltpu.VMEM((B, S, H), jnp.float32),
            pltpu.VMEM((T, E), jnp.bfloat16),
            pltpu.VMEM((NBUF, TV, E), jnp.float32),
            pltpu.SemaphoreType.DMA((NBUF,)),
        ],
    )(x_tm, wih0t, b0, whh0t, whh1t_cat, b1, wp1t, bp1, wp2t, bp2,
      emb_table, gen_b2d)


def kernel(sentence, emb_table, W_ih0, W_hh0, b_ih0, b_hh0, W_ih1, W_hh1,
           b_ih1, b_hh1, W_p1, b_p1, W_p2, b_p2, gen_b):
    idx_tm = jnp.transpose(sentence).reshape(T).astype(jnp.int32)
    x_tm = _sc_gather(emb_table, idx_tm)

    whh1t_cat = jnp.concatenate(
        [W_ih1.T, W_hh1.T], axis=0).astype(jnp.bfloat16)

    logits = _fused(
        x_tm,
        W_ih0.T.astype(jnp.bfloat16), (b_ih0 + b_hh0).reshape(1, G),
        W_hh0.T.astype(jnp.bfloat16), whh1t_cat,
        (b_ih1 + b_hh1).reshape(1, G),
        W_p1.T.astype(jnp.bfloat16), b_p1.reshape(1, H),
        W_p2.T.astype(jnp.bfloat16), b_p2.reshape(1, E),
        emb_table, gen_b.reshape(1, V))
    return logits.reshape(B, S, V)


# speedup vs baseline: 1.0733x; 1.0161x over previous
import functools

import jax
import jax.numpy as jnp
from jax import lax
from jax.experimental import pallas as pl
from jax.experimental.pallas import tpu as pltpu
from jax.experimental.pallas import tpu_sc as plsc

V = 100000
E = 128
H = 512
B = 8
S = 64
T = B * S
G = 4 * H

TV = 4096
NV = (V + TV - 1) // TV
LAST = V - (NV - 1) * TV
NBUF = 14


def _sc_gather(table, idx_flat):
    info = plsc.get_sparse_core_info()
    nc, ns = info.num_cores, info.num_subcores
    nw = nc * ns
    bpw = T // nw
    mesh = plsc.VectorSubcoreMesh(core_axis_name="c", subcore_axis_name="s")

    @functools.partial(
        pl.kernel,
        mesh=mesh,
        out_type=jax.ShapeDtypeStruct((T, E), jnp.float32),
        scratch_types=[
            pltpu.VMEM((bpw,), jnp.int32),
            pltpu.VMEM((bpw, E), jnp.float32),
            pltpu.SemaphoreType.DMA,
        ],
    )
    def k(table_hbm, idx_hbm, out_hbm, idx_v, rows_v, sem):
        wid = lax.axis_index("s") * nc + lax.axis_index("c")
        base = wid * bpw
        pltpu.sync_copy(idx_hbm.at[pl.ds(base, bpw)], idx_v)
        pltpu.async_copy(table_hbm.at[idx_v], rows_v, sem).wait()
        pltpu.sync_copy(rows_v, out_hbm.at[pl.ds(base, bpw)])

    return k(table, idx_flat)


def _run_lstm(x_ref, wih0t, b0, whh0t, whh1t_cat, b1, wp1t, bp1, wp2t,
              bp2, pre0_ref, hs_ref, outs_ref):
    pre0_ref[...] = b0[...] + jnp.dot(
        x_ref[...].astype(jnp.bfloat16), wih0t[...],
        preferred_element_type=jnp.float32)

    def step(k, carry):
        h0, c0, h1, c1 = carry
        h0b = h0.astype(jnp.bfloat16)
        a1 = jnp.concatenate([h0b, h1.astype(jnp.bfloat16)], axis=1)
        row = jnp.minimum(k, S - 1) * B
        g0 = pre0_ref[pl.ds(row, B), :] + jnp.dot(
            h0b, whh0t[...], preferred_element_type=jnp.float32)
        g1 = b1[...] + jnp.dot(
            a1, whh1t_cat[...], preferred_element_type=jnp.float32)
        i0 = jax.nn.sigmoid(g0[:, 0:H])
        f0 = jax.nn.sigmoid(g0[:, H:2 * H])
        t0 = jnp.tanh(g0[:, 2 * H:3 * H])
        o0 = jax.nn.sigmoid(g0[:, 3 * H:4 * H])
        i1 = jax.nn.sigmoid(g1[:, 0:H])
        f1 = jax.nn.sigmoid(g1[:, H:2 * H])
        t1 = jnp.tanh(g1[:, 2 * H:3 * H])
        o1 = jax.nn.sigmoid(g1[:, 3 * H:4 * H])
        c0 = f0 * c0 + i0 * t0
        h0 = o0 * jnp.tanh(c0)
        c1n = f1 * c1 + i1 * t1
        h1n = o1 * jnp.tanh(c1n)
        first = k == 0
        z = jnp.zeros((B, H), jnp.float32)
        c1 = jnp.where(first, z, c1n)
        h1 = jnp.where(first, z, h1n)
        hs_ref[:, jnp.maximum(k - 1, 0), :] = h1
        return h0, c0, h1, c1

    z = jnp.zeros((B, H), jnp.float32)
    lax.fori_loop(0, S + 1, step, (z, z, z, z), unroll=13)

    p1 = jnp.tanh(
        jnp.dot(hs_ref[...].reshape(T, H).astype(jnp.bfloat16), wp1t[...],
                preferred_element_type=jnp.float32)
        + bp1[...])
    outs_ref[...] = (jnp.dot(
        p1.astype(jnp.bfloat16), wp2t[...],
        preferred_element_type=jnp.float32) + bp2[...]).astype(jnp.bfloat16)


def _fused_body(x_ref, wih0t, b0, whh0t, whh1t_cat, b1, wp1t, bp1, wp2t,
                bp2, emb_hbm, gb_ref, out_ref,
                pre0_ref, hs_ref, outs_ref, ring_ref, sems):
    i = pl.program_id(0)

    @pl.when(i == 0)
    def _prologue():
        for s in range(NBUF):
            pltpu.make_async_copy(
                emb_hbm.at[pl.ds(s * TV, TV), :], ring_ref.at[s],
                sems.at[s]).start()
        _run_lstm(x_ref, wih0t, b0, whh0t, whh1t_cat, b1, wp1t, bp1,
                  wp2t, bp2, pre0_ref, hs_ref, outs_ref)

    @pl.when(jnp.logical_and(i >= 1, i <= NV - NBUF - 1))
    def _refill():
        j = i - 1 + NBUF
        slot = lax.rem(i - 1, NBUF)
        pltpu.make_async_copy(
            emb_hbm.at[pl.ds(j * TV, TV), :], ring_ref.at[slot],
            sems.at[slot]).start()

    @pl.when(i == NV - NBUF)
    def _refill_last():
        slot = lax.rem(i - 1, NBUF)
        pltpu.make_async_copy(
            emb_hbm.at[pl.ds((NV - 1) * TV, LAST), :],
            ring_ref.at[slot, pl.ds(0, LAST), :], sems.at[slot]).start()

    slot = lax.rem(i, NBUF)

    @pl.when(i < NV - 1)
    def _wait_full():
        pltpu.make_async_copy(
            emb_hbm.at[pl.ds(i * TV, TV), :], ring_ref.at[slot],
            sems.at[slot]).wait()

    @pl.when(i == NV - 1)
    def _wait_last():
        pltpu.make_async_copy(
            emb_hbm.at[pl.ds((NV - 1) * TV, LAST), :],
            ring_ref.at[slot, pl.ds(0, LAST), :], sems.at[slot]).wait()

    out_ref[...] = lax.dot_general(
        outs_ref[...], ring_ref[slot].astype(jnp.bfloat16),
        (((1,), (1,)), ((), ())),
        preferred_element_type=jnp.float32,
    ) + gb_ref[...]


def _fused(x_tm, wih0t, b0, whh0t, whh1t_cat, b1, wp1t, bp1, wp2t, bp2,
           emb_table, gen_b2d):
    full = lambda shape: pl.BlockSpec(shape, lambda i: tuple(
        0 for _ in shape))
    return pl.pallas_call(
        _fused_body,
        grid=(NV,),
        in_specs=[
            full((T, E)),
            full((E, G)), full((1, G)),
            full((H, G)), full((2 * H, G)), full((1, G)),
            full((H, H)), full((1, H)),
            full((H, E)), full((1, E)),
            pl.BlockSpec(memory_space=pl.ANY),
            pl.BlockSpec((1, TV), lambda i: (0, i)),
        ],
        out_specs=pl.BlockSpec((T, TV), lambda i: (0, i)),
        out_shape=jax.ShapeDtypeStruct((T, V), jnp.float32),
        scratch_shapes=[
            pltpu.VMEM((T, G), jnp.float32),
            pltpu.VMEM((B, S, H), jnp.float32),
            pltpu.VMEM((T, E), jnp.bfloat16),
            pltpu.VMEM((NBUF, TV, E), jnp.float32),
            pltpu.SemaphoreType.DMA((NBUF,)),
        ],
    )(x_tm, wih0t, b0, whh0t, whh1t_cat, b1, wp1t, bp1, wp2t, bp2,
      emb_table, gen_b2d)


def kernel(sentence, emb_table, W_ih0, W_hh0, b_ih0, b_hh0, W_ih1, W_hh1,
           b_ih1, b_hh1, W_p1, b_p1, W_p2, b_p2, gen_b):
    idx_tm = jnp.transpose(sentence).reshape(T).astype(jnp.int32)
    x_tm = _sc_gather(emb_table, idx_tm)

    whh1t_cat = jnp.concatenate(
        [W_ih1.T, W_hh1.T], axis=0).astype(jnp.bfloat16)

    logits = _fused(
        x_tm,
        W_ih0.T.astype(jnp.bfloat16), (b_ih0 + b_hh0).reshape(1, G),
        W_hh0.T.astype(jnp.bfloat16), whh1t_cat,
        (b_ih1 + b_hh1).reshape(1, G),
        W_p1.T.astype(jnp.bfloat16), b_p1.reshape(1, H),
        W_p2.T.astype(jnp.bfloat16), b_p2.reshape(1, E),
        emb_table, gen_b.reshape(1, V))
    return logits.reshape(B, S, V)


# speedup vs baseline: 1.0752x; 1.0018x over previous
import functools

import jax
import jax.numpy as jnp
from jax import lax
from jax.experimental import pallas as pl
from jax.experimental.pallas import tpu as pltpu
from jax.experimental.pallas import tpu_sc as plsc

V = 100000
E = 128
H = 512
B = 8
S = 64
T = B * S
G = 4 * H

TV = 4096
NV = (V + TV - 1) // TV
LAST = V - (NV - 1) * TV
NBUF = 15


def _sc_gather(table, idx_flat):
    info = plsc.get_sparse_core_info()
    nc, ns = info.num_cores, info.num_subcores
    nw = nc * ns
    bpw = T // nw
    mesh = plsc.VectorSubcoreMesh(core_axis_name="c", subcore_axis_name="s")

    @functools.partial(
        pl.kernel,
        mesh=mesh,
        out_type=jax.ShapeDtypeStruct((T, E), jnp.float32),
        scratch_types=[
            pltpu.VMEM((bpw,), jnp.int32),
            pltpu.VMEM((bpw, E), jnp.float32),
            pltpu.SemaphoreType.DMA,
        ],
    )
    def k(table_hbm, idx_hbm, out_hbm, idx_v, rows_v, sem):
        wid = lax.axis_index("s") * nc + lax.axis_index("c")
        base = wid * bpw
        pltpu.sync_copy(idx_hbm.at[pl.ds(base, bpw)], idx_v)
        pltpu.async_copy(table_hbm.at[idx_v], rows_v, sem).wait()
        pltpu.sync_copy(rows_v, out_hbm.at[pl.ds(base, bpw)])

    return k(table, idx_flat)


def _run_lstm(x_ref, wih0t, b0, whh0t, whh1t_cat, b1, wp1t, bp1, wp2t,
              bp2, pre0_ref, hs_ref, outs_ref):
    pre0_ref[...] = (b0[...] + jnp.dot(
        x_ref[...].astype(jnp.bfloat16), wih0t[...],
        preferred_element_type=jnp.float32)).astype(jnp.bfloat16)

    def step(k, carry):
        h0, c0, h1, c1 = carry
        h0b = h0.astype(jnp.bfloat16)
        a1 = jnp.concatenate([h0b, h1.astype(jnp.bfloat16)], axis=1)
        row = jnp.minimum(k, S - 1) * B
        g0 = pre0_ref[pl.ds(row, B), :].astype(jnp.float32) + jnp.dot(
            h0b, whh0t[...], preferred_element_type=jnp.float32)
        g1 = b1[...] + jnp.dot(
            a1, whh1t_cat[...], preferred_element_type=jnp.float32)
        i0 = jax.nn.sigmoid(g0[:, 0:H])
        f0 = jax.nn.sigmoid(g0[:, H:2 * H])
        t0 = jnp.tanh(g0[:, 2 * H:3 * H])
        o0 = jax.nn.sigmoid(g0[:, 3 * H:4 * H])
        i1 = jax.nn.sigmoid(g1[:, 0:H])
        f1 = jax.nn.sigmoid(g1[:, H:2 * H])
        t1 = jnp.tanh(g1[:, 2 * H:3 * H])
        o1 = jax.nn.sigmoid(g1[:, 3 * H:4 * H])
        c0 = f0 * c0 + i0 * t0
        h0 = o0 * jnp.tanh(c0)
        c1n = f1 * c1 + i1 * t1
        h1n = o1 * jnp.tanh(c1n)
        first = k == 0
        z = jnp.zeros((B, H), jnp.float32)
        c1 = jnp.where(first, z, c1n)
        h1 = jnp.where(first, z, h1n)
        hs_ref[:, jnp.maximum(k - 1, 0), :] = h1
        return h0, c0, h1, c1

    z = jnp.zeros((B, H), jnp.float32)
    lax.fori_loop(0, S + 1, step, (z, z, z, z), unroll=13)

    p1 = jnp.tanh(
        jnp.dot(hs_ref[...].reshape(T, H).astype(jnp.bfloat16), wp1t[...],
                preferred_element_type=jnp.float32)
        + bp1[...])
    outs_ref[...] = (jnp.dot(
        p1.astype(jnp.bfloat16), wp2t[...],
        preferred_element_type=jnp.float32) + bp2[...]).astype(jnp.bfloat16)


def _fused_body(x_ref, wih0t, b0, whh0t, whh1t_cat, b1, wp1t, bp1, wp2t,
                bp2, emb_hbm, gb_ref, out_ref,
                pre0_ref, hs_ref, outs_ref, ring_ref, sems):
    i = pl.program_id(0)

    @pl.when(i == 0)
    def _prologue():
        for s in range(NBUF):
            pltpu.make_async_copy(
                emb_hbm.at[pl.ds(s * TV, TV), :], ring_ref.at[s],
                sems.at[s]).start()
        _run_lstm(x_ref, wih0t, b0, whh0t, whh1t_cat, b1, wp1t, bp1,
                  wp2t, bp2, pre0_ref, hs_ref, outs_ref)

    @pl.when(jnp.logical_and(i >= 1, i <= NV - NBUF - 1))
    def _refill():
        j = i - 1 + NBUF
        slot = lax.rem(i - 1, NBUF)
        pltpu.make_async_copy(
            emb_hbm.at[pl.ds(j * TV, TV), :], ring_ref.at[slot],
            sems.at[slot]).start()

    @pl.when(i == NV - NBUF)
    def _refill_last():
        slot = lax.rem(i - 1, NBUF)
        pltpu.make_async_copy(
            emb_hbm.at[pl.ds((NV - 1) * TV, LAST), :],
            ring_ref.at[slot, pl.ds(0, LAST), :], sems.at[slot]).start()

    slot = lax.rem(i, NBUF)

    @pl.when(i < NV - 1)
    def _wait_full():
        pltpu.make_async_copy(
            emb_hbm.at[pl.ds(i * TV, TV), :], ring_ref.at[slot],
            sems.at[slot]).wait()

    @pl.when(i == NV - 1)
    def _wait_last():
        pltpu.make_async_copy(
            emb_hbm.at[pl.ds((NV - 1) * TV, LAST), :],
            ring_ref.at[slot, pl.ds(0, LAST), :], sems.at[slot]).wait()

    out_ref[...] = lax.dot_general(
        outs_ref[...], ring_ref[slot].astype(jnp.bfloat16),
        (((1,), (1,)), ((), ())),
        preferred_element_type=jnp.float32,
    ) + gb_ref[...]


def _fused(x_tm, wih0t, b0, whh0t, whh1t_cat, b1, wp1t, bp1, wp2t, bp2,
           emb_table, gen_b2d):
    full = lambda shape: pl.BlockSpec(shape, lambda i: tuple(
        0 for _ in shape))
    return pl.pallas_call(
        _fused_body,
        grid=(NV,),
        in_specs=[
            full((T, E)),
            full((E, G)), full((1, G)),
            full((H, G)), full((2 * H, G)), full((1, G)),
            full((H, H)), full((1, H)),
            full((H, E)), full((1, E)),
            pl.BlockSpec(memory_space=pl.ANY),
            pl.BlockSpec((1, TV), lambda i: (0, i)),
        ],
        out_specs=pl.BlockSpec((T, TV), lambda i: (0, i)),
        out_shape=jax.ShapeDtypeStruct((T, V), jnp.float32),
        scratch_shapes=[
            pltpu.VMEM((T, G), jnp.bfloat16),
            pltpu.VMEM((B, S, H), jnp.float32),
            pltpu.VMEM((T, E), jnp.bfloat16),
            pltpu.VMEM((NBUF, TV, E), jnp.float32),
            pltpu.SemaphoreType.DMA((NBUF,)),
        ],
    )(x_tm, wih0t, b0, whh0t, whh1t_cat, b1, wp1t, bp1, wp2t, bp2,
      emb_table, gen_b2d)


def kernel(sentence, emb_table, W_ih0, W_hh0, b_ih0, b_hh0, W_ih1, W_hh1,
           b_ih1, b_hh1, W_p1, b_p1, W_p2, b_p2, gen_b):
    idx_tm = jnp.transpose(sentence).reshape(T).astype(jnp.int32)
    x_tm = _sc_gather(emb_table, idx_tm)

    whh1t_cat = jnp.concatenate(
        [W_ih1.T, W_hh1.T], axis=0).astype(jnp.bfloat16)

    logits = _fused(
        x_tm,
        W_ih0.T.astype(jnp.bfloat16), (b_ih0 + b_hh0).reshape(1, G),
        W_hh0.T.astype(jnp.bfloat16), whh1t_cat,
        (b_ih1 + b_hh1).reshape(1, G),
        W_p1.T.astype(jnp.bfloat16), b_p1.reshape(1, H),
        W_p2.T.astype(jnp.bfloat16), b_p2.reshape(1, E),
        emb_table, gen_b.reshape(1, V))
    return logits.reshape(B, S, V)
